# Initial kernel scaffold; baseline (speedup 1.0000x reference)
import jax
import jax.numpy as jnp
from jax.experimental import pallas as pl


def kernel(box_pred, cls_pred):
    raise NotImplementedError("write your pallas kernel here")


# speedup vs baseline: 1.4458x; 1.4458x over previous
import functools

import jax
import jax.numpy as jnp
import numpy as np
from jax.experimental import pallas as pl

_IMG_H = 512
_IMG_W = 512
_NUM_CLASSES = 20
_BATCH = 2
_MAX_TOTAL = 200
_SCORE_THR = 0.5
_IOU_THR = 0.5
_PRE_NMS = 2000
_N_ANCHORS = 49104
_VARIANCES = np.array([0.1, 0.1, 0.2, 0.2], dtype=np.float32)


def _make_anchors(image_h, image_w):
    aspect_ratios = [0.5, 1.0, 2.0]
    scales = [2.0 ** x for x in [0.0, 1.0 / 3.0, 2.0 / 3.0]]
    areas = [32.0 ** 2, 64.0 ** 2, 128.0 ** 2, 256.0 ** 2, 512.0 ** 2]
    strides = [2 ** i for i in range(3, 8)]
    all_anchors = []
    for area, stride in zip(areas, strides):
        dims = []
        for ratio in aspect_ratios:
            ah = np.sqrt(area / ratio)
            aw = area / ah
            for scale in scales:
                dims.append([aw * scale, ah * scale])
        dims = np.array(dims, dtype=np.float32)
        fh = int(np.ceil(image_h / stride))
        fw = int(np.ceil(image_w / stride))
        rx = (np.arange(fw, dtype=np.float32) + 0.5) * stride
        ry = (np.arange(fh, dtype=np.float32) + 0.5) * stride
        cx, cy = np.meshgrid(rx, ry)
        centers = np.stack([cx, cy], axis=-1).astype(np.float32)[:, :, None, :]
        centers = np.tile(centers, (1, 1, 9, 1))
        d = np.broadcast_to(dims[None, None, :, :], (fh, fw, 9, 2))
        all_anchors.append(np.concatenate([centers, d], axis=-1).reshape(-1, 4))
    return np.concatenate(all_anchors, axis=0)


_ANCHORS = _make_anchors(_IMG_H, _IMG_W)


def _sigmoid_body(x_ref, o_ref):
    o_ref[...] = 1.0 / (1.0 + jnp.exp(-x_ref[...]))


def _iou_one_vs_all(box, boxes):
    lt = jnp.maximum(box[:2], boxes[:, :2])
    rb = jnp.minimum(box[2:], boxes[:, 2:])
    wh = jnp.maximum(rb - lt, 0.0)
    inter = wh[:, 0] * wh[:, 1]
    a1 = (box[2] - box[0]) * (box[3] - box[1])
    a2 = (boxes[:, 2] - boxes[:, 0]) * (boxes[:, 3] - boxes[:, 1])
    return inter / (a1 + a2 - inter + 1e-8)


def _nms_one_class(boxes, scores):
    top_scores, top_idx = jax.lax.top_k(scores, _PRE_NMS)
    top_boxes = boxes[top_idx]
    masked = jnp.where(top_scores >= _SCORE_THR, top_scores, -1.0)

    def body(i, carry):
        m, sb, ss = carry
        best = jnp.argmax(m)
        bscore = m[best]
        bbox = top_boxes[best]
        valid = bscore > 0.0
        ss = ss.at[i].set(jnp.where(valid, bscore, 0.0))
        sb = sb.at[i].set(jnp.where(valid, bbox, jnp.zeros((4,), dtype=boxes.dtype)))
        ious = _iou_one_vs_all(bbox, top_boxes)
        m = jnp.where(ious > _IOU_THR, -1.0, m)
        m = m.at[best].set(-1.0)
        return (m, sb, ss)

    init = (masked,
            jnp.zeros((_MAX_TOTAL, 4), dtype=boxes.dtype),
            jnp.zeros((_MAX_TOTAL,), dtype=scores.dtype))
    _, sel_boxes, sel_scores = jax.lax.fori_loop(0, _MAX_TOTAL, body, init)
    return sel_boxes, sel_scores


def _combined_nms(boxes, cls_scores):
    sb, ss = jax.vmap(lambda s: _nms_one_class(boxes, s))(jnp.transpose(cls_scores))
    labels = jnp.broadcast_to(
        jnp.arange(_NUM_CLASSES, dtype=jnp.float32)[:, None], ss.shape)
    flat_s = ss.reshape(-1)
    flat_b = sb.reshape(-1, 4)
    flat_l = labels.reshape(-1)
    top_s, idx = jax.lax.top_k(flat_s, _MAX_TOTAL)
    top_b = flat_b[idx]
    top_l = jnp.where(top_s > 0.0, flat_l[idx], 0.0)
    return top_b, top_s, top_l


@jax.jit
def _run(box_pred, cls_pred):
    anchors = jnp.asarray(_ANCHORS)
    variances = jnp.asarray(_VARIANCES)
    flat = cls_pred.reshape(-1, 128)
    cls_scores = pl.pallas_call(
        _sigmoid_body,
        out_shape=jax.ShapeDtypeStruct(flat.shape, flat.dtype),
    )(flat).reshape(cls_pred.shape)
    b = box_pred * variances
    xy = b[:, :, :2] * anchors[None, :, 2:] + anchors[None, :, :2]
    wh = jnp.exp(b[:, :, 2:]) * anchors[None, :, 2:]
    boxes = jnp.concatenate([xy - wh / 2.0, xy + wh / 2.0], axis=-1)
    tb, ts, tl = jax.vmap(_combined_nms)(boxes, cls_scores)
    return (tb, ts, tl)


def kernel(box_pred, cls_pred):
    return _run(box_pred, cls_pred)


# speedup vs baseline: 9.7287x; 6.7291x over previous
import functools

import jax
import jax.numpy as jnp
import numpy as np
from jax import lax
from jax.experimental import pallas as pl
from jax.experimental.pallas import tpu as pltpu
from jax.experimental.pallas import tpu_sc as plsc

_NUM_CLASSES = 20
_BATCH = 2
_MAX_TOTAL = 200
_SCORE_THR = 0.5
_IOU_THR = 0.5
_PRE_NMS = 2000
_N = 49104
_NPAD = 49152
_NPROB = _BATCH * _NUM_CLASSES
_CAND = 2048
_SEL = 256


def _make_anchors(image_h, image_w):
    aspect_ratios = [0.5, 1.0, 2.0]
    scales = [2.0 ** x for x in [0.0, 1.0 / 3.0, 2.0 / 3.0]]
    areas = [32.0 ** 2, 64.0 ** 2, 128.0 ** 2, 256.0 ** 2, 512.0 ** 2]
    strides = [2 ** i for i in range(3, 8)]
    all_anchors = []
    for area, stride in zip(areas, strides):
        dims = []
        for ratio in aspect_ratios:
            ah = np.sqrt(area / ratio)
            aw = area / ah
            for scale in scales:
                dims.append([aw * scale, ah * scale])
        dims = np.array(dims, dtype=np.float32)
        fh = int(np.ceil(image_h / stride))
        fw = int(np.ceil(image_w / stride))
        rx = (np.arange(fw, dtype=np.float32) + 0.5) * stride
        ry = (np.arange(fh, dtype=np.float32) + 0.5) * stride
        cx, cy = np.meshgrid(rx, ry)
        centers = np.stack([cx, cy], axis=-1).astype(np.float32)[:, :, None, :]
        centers = np.tile(centers, (1, 1, 9, 1))
        d = np.broadcast_to(dims[None, None, :, :], (fh, fw, 9, 2))
        all_anchors.append(np.concatenate([centers, d], axis=-1).reshape(-1, 4))
    return np.concatenate(all_anchors, axis=0)


_ANCH = np.zeros((4, _NPAD), dtype=np.float32)
_ANCH[:, :_N] = _make_anchors(512, 512).T
_ANCH[2:, _N:] = 1.0


_BLK = 512


def _tc_body(logits_ref, bp_ref, anch_ref, scores_ref, boxes_ref,
             thr_ref, quota_ref):
    nblk = _NPAD // _BLK

    def sig_blk(j, _):
        s = pl.ds(j * _BLK, _BLK)
        x = logits_ref[:, s]
        scores_ref[:, s] = 1.0 / (1.0 + jnp.exp(-x))
        return 0
    lax.fori_loop(0, nblk, sig_blk, 0, unroll=2)

    def dec_blk(j, _):
        s = pl.ds(j * _BLK, _BLK)
        bp = bp_ref[:, :, s]
        an = anch_ref[:, s]
        acx, acy, aw, ah = an[0], an[1], an[2], an[3]
        tx = bp[:, 0] * jnp.float32(0.1)
        ty = bp[:, 1] * jnp.float32(0.1)
        tw = bp[:, 2] * jnp.float32(0.2)
        th = bp[:, 3] * jnp.float32(0.2)
        cx = tx * aw + acx
        cy = ty * ah + acy
        w = jnp.exp(tw) * aw
        h = jnp.exp(th) * ah
        wh_x = w / 2.0
        wh_y = h / 2.0
        boxes_ref[:, 0, s] = cx - wh_x
        boxes_ref[:, 1, s] = cy - wh_y
        boxes_ref[:, 2, s] = cx + wh_x
        boxes_ref[:, 3, s] = cy + wh_y
        return 0
    lax.fori_loop(0, nblk, dec_blk, 0, unroll=2)

    def count_ge(test):
        def blk(j, acc):
            s = pl.ds(j * _BLK, _BLK)
            b = lax.bitcast_convert_type(scores_ref[:, s], jnp.int32)
            return acc + jnp.sum(
                jnp.where(b >= test, jnp.int32(1), jnp.int32(0)),
                axis=1, keepdims=True)
        return lax.fori_loop(0, nblk, blk, jnp.zeros((_NPROB, 1), jnp.int32),
                             unroll=2)

    def round_body(r, p):
        test = p + lax.shift_left(jnp.int32(1), jnp.int32(29) - r)
        cnt = count_ge(test)
        return jnp.where(cnt >= _PRE_NMS, test, p)

    pbits = lax.fori_loop(0, 30, round_body, jnp.zeros((_NPROB, 1), jnp.int32))
    cnt_gt = count_ge(pbits + 1)
    quota = _PRE_NMS - cnt_gt

    thr_ref[:, :] = lax.bitcast_convert_type(pbits, jnp.float32)
    quota_ref[:, :] = quota


def _run_tc(logits_t, bp_t, anch):
    return pl.pallas_call(
        _tc_body,
        out_shape=(
            jax.ShapeDtypeStruct((_NPROB, _NPAD), jnp.float32),
            jax.ShapeDtypeStruct((_BATCH, 4, _NPAD), jnp.float32),
            jax.ShapeDtypeStruct((_NPROB, 1), jnp.float32),
            jax.ShapeDtypeStruct((_NPROB, 1), jnp.int32),
        ),
    )(logits_t, bp_t, anch)


_mesh = plsc.VectorSubcoreMesh(core_axis_name="c", subcore_axis_name="s")
_L = 16
_NCH = _NPAD // _L
_CCH = _CAND // _L


def _sc_nms_problem(p, scores_hbm, boxes_hbm, thr_v, quota_v,
                    s_v, cidx_v, cs_v, cx1_v, cy1_v, cx2_v, cy2_v,
                    os_v, ob_v, sel_s_hbm, sel_b_hbm, sem):
    lanes = lax.iota(jnp.int32, _L)
    lane0 = lanes == 0
    p_vec = jnp.full((_L,), p, jnp.int32)
    b = p // _NUM_CLASSES

    pltpu.sync_copy(scores_hbm.at[p], s_v)
    thr = plsc.load_gather(thr_v, [p_vec])
    quota = plsc.load_gather(quota_v, [p_vec])

    def init_s(k, _):
        cs_v[pl.ds(k * _L, _L)] = jnp.full((_L,), -1.0, jnp.float32)
        return 0
    lax.fori_loop(0, _CCH, init_s, 0)
    def init_i(k, _):
        cidx_v[pl.ds(k * _L, _L)] = jnp.zeros((_L,), jnp.int32)
        return 0
    lax.fori_loop(0, _CCH, init_i, 0)

    def comp_blk(k, carry):
        ptr, ties = carry
        sl = pl.ds(k * _L, _L)
        s = s_v[sl]
        idxv = k * _L + lanes
        gt = s > thr
        eq = s == thr
        pref = jnp.cumsum(eq.astype(jnp.int32))
        tie_ok = eq & ((ties + pref - 1) < quota)
        m = (s >= _SCORE_THR) & (gt | tie_ok)
        pos = jnp.cumsum(m.astype(jnp.int32))
        dst = ptr + pos - 1
        plsc.store_scatter(cs_v, [dst], s, mask=m)
        plsc.store_scatter(cidx_v, [dst], idxv, mask=m)
        ptr = ptr + plsc.all_reduce_population_count(m)
        ties = ties + plsc.all_reduce_population_count(eq)
        return (ptr, ties)

    zero = jnp.zeros((_L,), jnp.int32)
    lax.fori_loop(0, _NCH, comp_blk, (zero, zero))

    for comp, dstv in enumerate((cx1_v, cy1_v, cx2_v, cy2_v)):
        pltpu.sync_copy(boxes_hbm.at[b, comp], s_v)
        def gat(k, _):
            sl = pl.ds(k * _L, _L)
            dstv[sl] = plsc.load_gather(s_v, [cidx_v[sl]])
            return 0
        lax.fori_loop(0, _CCH, gat, 0)

    def nms_pass(bx1, by1, bx2, by2):
        ba1 = (bx2 - bx1) * (by2 - by1)
        def blk(k, carry):
            rv, ri = carry
            sl = pl.ds(k * _L, _L)
            s = cs_v[sl]
            x1 = cx1_v[sl]; y1 = cy1_v[sl]; x2 = cx2_v[sl]; y2 = cy2_v[sl]
            ltx = jnp.maximum(bx1, x1); lty = jnp.maximum(by1, y1)
            rbx = jnp.minimum(bx2, x2); rby = jnp.minimum(by2, y2)
            w = jnp.maximum(rbx - ltx, 0.0); h = jnp.maximum(rby - lty, 0.0)
            inter = w * h
            a2 = (x2 - x1) * (y2 - y1)
            iou = inter / (ba1 + a2 - inter + jnp.float32(1e-8))
            s2 = jnp.where(iou > _IOU_THR, -1.0, s)
            cs_v[sl] = s2
            idxv = k * _L + lanes
            upd = s2 > rv
            ri = jnp.where(upd, idxv, ri)
            rv = jnp.maximum(rv, s2)
            return (rv, ri)
        rv0 = jnp.full((_L,), -2.0, jnp.float32)
        rv, ri = lax.fori_loop(0, _CCH, blk, (rv0, zero))
        mval = jnp.max(rv)
        midx = jnp.min(jnp.where(rv == mval, ri, jnp.int32(_CAND)))
        return mval, midx

    far = jnp.full((_L,), -1e6, jnp.float32)
    mval, midx = nms_pass(far, far, far, far)

    def init_o(k, _):
        sl = pl.ds(k * _L, _L)
        os_v[sl] = jnp.zeros((_L,), jnp.float32)
        for comp in range(4):
            ob_v[comp, sl] = jnp.zeros((_L,), jnp.float32)
        return 0
    lax.fori_loop(0, _SEL // _L, init_o, 0)

    def sel_body(i, carry):
        mval, midx = carry
        iv = jnp.full((_L,), midx, jnp.int32)
        valid = mval > 0.0
        bx1 = plsc.load_gather(cx1_v, [iv])
        by1 = plsc.load_gather(cy1_v, [iv])
        bx2 = plsc.load_gather(cx2_v, [iv])
        by2 = plsc.load_gather(cy2_v, [iv])
        tv = jnp.full((_L,), i, jnp.int32)
        sval = jnp.where(valid, mval, 0.0)
        plsc.store_scatter(os_v, [tv], jnp.full((_L,), sval, jnp.float32),
                           mask=lane0)
        vf = jnp.where(valid, 1.0, 0.0)
        plsc.store_scatter(ob_v, [jnp.zeros((_L,), jnp.int32), tv],
                           bx1 * vf, mask=lane0)
        plsc.store_scatter(ob_v, [jnp.full((_L,), 1, jnp.int32), tv],
                           by1 * vf, mask=lane0)
        plsc.store_scatter(ob_v, [jnp.full((_L,), 2, jnp.int32), tv],
                           bx2 * vf, mask=lane0)
        plsc.store_scatter(ob_v, [jnp.full((_L,), 3, jnp.int32), tv],
                           by2 * vf, mask=lane0)
        plsc.store_scatter(cs_v, [iv], jnp.full((_L,), -1.0, jnp.float32),
                           mask=lane0)
        return nms_pass(bx1, by1, bx2, by2)

    lax.fori_loop(0, _MAX_TOTAL, sel_body, (mval, midx))

    pltpu.sync_copy(os_v, sel_s_hbm.at[pl.ds(p * _SEL, _SEL)])
    for comp in range(4):
        pltpu.sync_copy(ob_v.at[comp],
                        sel_b_hbm.at[pl.ds((p * 4 + comp) * _SEL, _SEL)])


@functools.partial(
    pl.kernel,
    mesh=_mesh,
    out_type=(
        jax.ShapeDtypeStruct((_NPROB * _SEL,), jnp.float32),
        jax.ShapeDtypeStruct((_NPROB * 4 * _SEL,), jnp.float32),
    ),
    compiler_params=pltpu.CompilerParams(needs_layout_passes=False),
    scratch_types=[
        pltpu.VMEM((128,), jnp.float32),
        pltpu.VMEM((128,), jnp.int32),
        pltpu.VMEM((_NPAD,), jnp.float32),
        pltpu.VMEM((_CAND,), jnp.int32),
        pltpu.VMEM((_CAND,), jnp.float32),
        pltpu.VMEM((_CAND,), jnp.float32),
        pltpu.VMEM((_CAND,), jnp.float32),
        pltpu.VMEM((_CAND,), jnp.float32),
        pltpu.VMEM((_CAND,), jnp.float32),
        pltpu.VMEM((_SEL,), jnp.float32),
        pltpu.VMEM((4, _SEL), jnp.float32),
        pltpu.SemaphoreType.DMA,
    ],
)
def _sc_nms(scores_hbm, boxes_hbm, thr_hbm, quota_hbm,
            sel_s_hbm, sel_b_hbm,
            thr_v, quota_v, s_v, cidx_v, cs_v, cx1_v, cy1_v, cx2_v, cy2_v,
            os_v, ob_v, sem):
    wid = lax.axis_index("s") * 2 + lax.axis_index("c")
    pltpu.sync_copy(thr_hbm, thr_v)
    pltpu.sync_copy(quota_hbm, quota_v)
    args = (scores_hbm, boxes_hbm, thr_v, quota_v, s_v, cidx_v, cs_v,
            cx1_v, cy1_v, cx2_v, cy2_v, os_v, ob_v, sel_s_hbm, sel_b_hbm, sem)
    _sc_nms_problem(wid, *args)

    @pl.when(wid < _NPROB - 32)
    def _():
        _sc_nms_problem(wid + 32, *args)


@functools.partial(
    pl.kernel,
    mesh=_mesh,
    out_type=(
        jax.ShapeDtypeStruct((_BATCH * _SEL,), jnp.float32),
        jax.ShapeDtypeStruct((_BATCH * 4 * _SEL,), jnp.float32),
        jax.ShapeDtypeStruct((_BATCH * _SEL,), jnp.float32),
    ),
    compiler_params=pltpu.CompilerParams(needs_layout_passes=False),
    scratch_types=[
        pltpu.VMEM((_NUM_CLASSES * _SEL,), jnp.float32),
        pltpu.VMEM((_NUM_CLASSES * 4 * _SEL,), jnp.float32),
        pltpu.VMEM((_SEL,), jnp.float32),
        pltpu.VMEM((4, _SEL), jnp.float32),
        pltpu.VMEM((_SEL,), jnp.float32),
    ],
)
def _sc_merge(sel_s_hbm, sel_b_hbm, out_s_hbm, out_b_hbm, out_l_hbm,
              ss_v, sb_v, os_v, ob_v, ol_v):
    wid = lax.axis_index("s") * 2 + lax.axis_index("c")
    lanes = lax.iota(jnp.int32, _L)
    lane0 = lanes == 0
    big = jnp.int32(9999)

    @pl.when(wid < _BATCH)
    def _():
        b = wid
        pltpu.sync_copy(
            sel_s_hbm.at[pl.ds(b * _NUM_CLASSES * _SEL, _NUM_CLASSES * _SEL)],
            ss_v)
        pltpu.sync_copy(
            sel_b_hbm.at[pl.ds(b * _NUM_CLASSES * 4 * _SEL,
                               _NUM_CLASSES * 4 * _SEL)],
            sb_v)

        def init_o(k, _):
            sl = pl.ds(k * _L, _L)
            os_v[sl] = jnp.zeros((_L,), jnp.float32)
            ol_v[sl] = jnp.zeros((_L,), jnp.float32)
            for comp in range(4):
                ob_v[comp, sl] = jnp.zeros((_L,), jnp.float32)
            return 0
        lax.fori_loop(0, _SEL // _L, init_o, 0)

        cls0 = lanes
        cls1 = _L + lanes
        live1 = cls1 < _NUM_CLASSES

        def step(t, carry):
            h0, h1 = carry
            v0 = plsc.load_gather(
                ss_v, [cls0 * _SEL + jnp.minimum(h0, _SEL - 1)])
            v1 = plsc.load_gather(
                ss_v, [jnp.minimum(cls1, _NUM_CLASSES - 1) * _SEL
                       + jnp.minimum(h1, _SEL - 1)])
            v0 = jnp.where(h0 >= _MAX_TOTAL, -1.0, v0)
            v1 = jnp.where(live1 & (h1 < _MAX_TOTAL), v1, -1.0)
            m0 = jnp.max(v0)
            m1 = jnp.max(v1)
            mval = jnp.maximum(m0, m1)
            c0 = jnp.min(jnp.where(v0 == mval, cls0, big))
            c1 = jnp.min(jnp.where(v1 == mval, cls1, big))
            cls = jnp.minimum(c0, c1)
            p0 = jnp.min(jnp.where((v0 == mval) & (cls0 == cls), h0, big))
            p1 = jnp.min(jnp.where((v1 == mval) & (cls1 == cls), h1, big))
            pos = jnp.minimum(p0, p1)

            cv = jnp.full((_L,), cls, jnp.int32)
            pv = jnp.full((_L,), pos, jnp.int32)
            tv = jnp.full((_L,), t, jnp.int32)
            plsc.store_scatter(os_v, [tv], jnp.full((_L,), mval, jnp.float32),
                               mask=lane0)
            lab = jnp.where(mval > 0.0, cls.astype(jnp.float32), 0.0)
            plsc.store_scatter(ol_v, [tv], jnp.full((_L,), lab, jnp.float32),
                               mask=lane0)
            for comp in range(4):
                bv = plsc.load_gather(
                    sb_v, [cv * (4 * _SEL) + comp * _SEL + pv])
                plsc.store_scatter(ob_v,
                                   [jnp.full((_L,), comp, jnp.int32), tv],
                                   bv, mask=lane0)
            adv0 = (cls0 == cls)
            adv1 = (cls1 == cls)
            h0 = h0 + jnp.where(adv0, 1, 0)
            h1 = h1 + jnp.where(adv1, 1, 0)
            return (h0, h1)

        zero = jnp.zeros((_L,), jnp.int32)
        lax.fori_loop(0, _MAX_TOTAL, step, (zero, zero))

        pltpu.sync_copy(os_v, out_s_hbm.at[pl.ds(b * _SEL, _SEL)])
        pltpu.sync_copy(ol_v, out_l_hbm.at[pl.ds(b * _SEL, _SEL)])
        for comp in range(4):
            pltpu.sync_copy(
                ob_v.at[comp],
                out_b_hbm.at[pl.ds((b * 4 + comp) * _SEL, _SEL)])


@jax.jit
def _run(box_pred, cls_pred):
    anch = jnp.asarray(_ANCH)
    logits_t = jnp.transpose(cls_pred, (0, 2, 1)).reshape(_NPROB, _N)
    logits_t = jnp.pad(logits_t, ((0, 0), (0, _NPAD - _N)),
                       constant_values=-1e9)
    bp_t = jnp.transpose(box_pred, (0, 2, 1))
    bp_t = jnp.pad(bp_t, ((0, 0), (0, 0), (0, _NPAD - _N)))

    scores, boxes, thr, quota = _run_tc(logits_t, bp_t, anch)
    thr_pad = jnp.pad(thr[:, 0], (0, 128 - _NPROB))
    quota_pad = jnp.pad(quota[:, 0], (0, 128 - _NPROB))
    sel_s, sel_b = _sc_nms(scores, boxes, thr_pad, quota_pad)
    out_s, out_b, out_l = _sc_merge(sel_s, sel_b)

    ts = out_s.reshape(_BATCH, _SEL)[:, :_MAX_TOTAL]
    tl = out_l.reshape(_BATCH, _SEL)[:, :_MAX_TOTAL]
    tb = jnp.transpose(out_b.reshape(_BATCH, 4, _SEL),
                       (0, 2, 1))[:, :_MAX_TOTAL, :]
    return (tb, ts, tl)


def kernel(box_pred, cls_pred):
    return _run(box_pred, cls_pred)


# speedup vs baseline: 21.3471x; 2.1942x over previous
import functools

import jax
import jax.numpy as jnp
import numpy as np
from jax import lax
from jax.experimental import pallas as pl
from jax.experimental.pallas import tpu as pltpu
from jax.experimental.pallas import tpu_sc as plsc

_NUM_CLASSES = 20
_BATCH = 2
_MAX_TOTAL = 200
_SCORE_THR = 0.5
_IOU_THR = 0.5
_PRE_NMS = 2000
_N = 49104
_NPAD = 49152
_NPROB = _BATCH * _NUM_CLASSES
_CAND = 2048
_SEL = 256


def _make_anchors(image_h, image_w):
    aspect_ratios = [0.5, 1.0, 2.0]
    scales = [2.0 ** x for x in [0.0, 1.0 / 3.0, 2.0 / 3.0]]
    areas = [32.0 ** 2, 64.0 ** 2, 128.0 ** 2, 256.0 ** 2, 512.0 ** 2]
    strides = [2 ** i for i in range(3, 8)]
    all_anchors = []
    for area, stride in zip(areas, strides):
        dims = []
        for ratio in aspect_ratios:
            ah = np.sqrt(area / ratio)
            aw = area / ah
            for scale in scales:
                dims.append([aw * scale, ah * scale])
        dims = np.array(dims, dtype=np.float32)
        fh = int(np.ceil(image_h / stride))
        fw = int(np.ceil(image_w / stride))
        rx = (np.arange(fw, dtype=np.float32) + 0.5) * stride
        ry = (np.arange(fh, dtype=np.float32) + 0.5) * stride
        cx, cy = np.meshgrid(rx, ry)
        centers = np.stack([cx, cy], axis=-1).astype(np.float32)[:, :, None, :]
        centers = np.tile(centers, (1, 1, 9, 1))
        d = np.broadcast_to(dims[None, None, :, :], (fh, fw, 9, 2))
        all_anchors.append(np.concatenate([centers, d], axis=-1).reshape(-1, 4))
    return np.concatenate(all_anchors, axis=0)


_ANCH = np.zeros((4, _NPAD), dtype=np.float32)
_ANCH[:, :_N] = _make_anchors(512, 512).T
_ANCH[2:, _N:] = 1.0


_BLK = 512


def _tc_body(logits_ref, bp_ref, anch_ref, scores_ref, boxes_ref,
             thr_ref, quota_ref):
    nblk = _NPAD // _BLK

    def sig_blk(j, _):
        s = pl.ds(j * _BLK, _BLK)
        x = logits_ref[:, s]
        scores_ref[:, s] = 1.0 / (1.0 + jnp.exp(-x))
        return 0
    lax.fori_loop(0, nblk, sig_blk, 0, unroll=2)

    def dec_blk(j, _):
        s = pl.ds(j * _BLK, _BLK)
        bp = bp_ref[:, :, s]
        an = anch_ref[:, s]
        acx, acy, aw, ah = an[0], an[1], an[2], an[3]
        tx = bp[:, 0] * jnp.float32(0.1)
        ty = bp[:, 1] * jnp.float32(0.1)
        tw = bp[:, 2] * jnp.float32(0.2)
        th = bp[:, 3] * jnp.float32(0.2)
        cx = tx * aw + acx
        cy = ty * ah + acy
        w = jnp.exp(tw) * aw
        h = jnp.exp(th) * ah
        wh_x = w / 2.0
        wh_y = h / 2.0
        boxes_ref[:, 0, s] = cx - wh_x
        boxes_ref[:, 1, s] = cy - wh_y
        boxes_ref[:, 2, s] = cx + wh_x
        boxes_ref[:, 3, s] = cy + wh_y
        return 0
    lax.fori_loop(0, nblk, dec_blk, 0, unroll=2)

    def count_ge(test):
        def blk(j, acc):
            s = pl.ds(j * _BLK, _BLK)
            b = lax.bitcast_convert_type(scores_ref[:, s], jnp.int32)
            return acc + jnp.sum(
                jnp.where(b >= test, jnp.int32(1), jnp.int32(0)),
                axis=1, keepdims=True)
        return lax.fori_loop(0, nblk, blk, jnp.zeros((_NPROB, 1), jnp.int32),
                             unroll=2)

    def round_body(r, p):
        test = p + lax.shift_left(jnp.int32(1), jnp.int32(29) - r)
        cnt = count_ge(test)
        return jnp.where(cnt >= _PRE_NMS, test, p)

    pbits = lax.fori_loop(0, 30, round_body, jnp.zeros((_NPROB, 1), jnp.int32))
    cnt_gt = count_ge(pbits + 1)
    quota = _PRE_NMS - cnt_gt

    thr_ref[:, :] = lax.bitcast_convert_type(pbits, jnp.float32)
    quota_ref[:, :] = quota


def _run_tc(logits_t, bp_t, anch):
    return pl.pallas_call(
        _tc_body,
        out_shape=(
            jax.ShapeDtypeStruct((_NPROB, _NPAD), jnp.float32),
            jax.ShapeDtypeStruct((_BATCH, 4, _NPAD), jnp.float32),
            jax.ShapeDtypeStruct((_NPROB, 1), jnp.float32),
            jax.ShapeDtypeStruct((_NPROB, 1), jnp.int32),
        ),
    )(logits_t, bp_t, anch)


_mesh = plsc.VectorSubcoreMesh(core_axis_name="c", subcore_axis_name="s")
_L = 16
_NCH = _NPAD // _L
_CCH = _CAND // _L


def _sc_nms_problem(p, scores_hbm, boxes_hbm, thr_v, quota_v,
                    s_v, cidx_v, cs_v, cx1_v, cy1_v, cx2_v, cy2_v,
                    ca_v, os_v, ob_v, sel_s_hbm, sel_b_hbm, sem):
    lanes = lax.iota(jnp.int32, _L)
    lane0 = lanes == 0
    p_vec = jnp.full((_L,), p, jnp.int32)
    b = p // _NUM_CLASSES

    pltpu.sync_copy(scores_hbm.at[p], s_v)
    thr = plsc.load_gather(thr_v, [p_vec])
    quota = plsc.load_gather(quota_v, [p_vec])

    def init_s(k, _):
        cs_v[pl.ds(k * _L, _L)] = jnp.full((_L,), -1.0, jnp.float32)
        return 0
    lax.fori_loop(0, _CCH, init_s, 0, unroll=4)
    def init_i(k, _):
        cidx_v[pl.ds(k * _L, _L)] = jnp.zeros((_L,), jnp.int32)
        return 0
    lax.fori_loop(0, _CCH, init_i, 0, unroll=4)

    def comp_blk(k, carry):
        ptr, ties = carry
        sl = pl.ds(k * _L, _L)
        s = s_v[sl]
        idxv = k * _L + lanes
        gt = s > thr
        eq = s == thr
        pref = jnp.cumsum(eq.astype(jnp.int32))
        tie_ok = eq & ((ties + pref - 1) < quota)
        m = (s >= _SCORE_THR) & (gt | tie_ok)
        pos = jnp.cumsum(m.astype(jnp.int32))
        dst = ptr + pos - 1
        plsc.store_scatter(cs_v, [dst], s, mask=m)
        plsc.store_scatter(cidx_v, [dst], idxv, mask=m)
        ptr = ptr + plsc.all_reduce_population_count(m)
        ties = ties + plsc.all_reduce_population_count(eq)
        return (ptr, ties)

    zero = jnp.zeros((_L,), jnp.int32)
    lax.fori_loop(0, _NCH, comp_blk, (zero, zero), unroll=4)

    for comp, dstv in enumerate((cx1_v, cy1_v, cx2_v, cy2_v)):
        pltpu.sync_copy(boxes_hbm.at[b, comp], s_v)
        def gat(k, _):
            sl = pl.ds(k * _L, _L)
            dstv[sl] = plsc.load_gather(s_v, [cidx_v[sl]])
            return 0
        lax.fori_loop(0, _CCH, gat, 0, unroll=4)

    def area_blk(k, _):
        sl = pl.ds(k * _L, _L)
        ca_v[sl] = (cx2_v[sl] - cx1_v[sl]) * (cy2_v[sl] - cy1_v[sl])
        return 0
    lax.fori_loop(0, _CCH, area_blk, 0, unroll=4)

    def nms_pass(bx1, by1, bx2, by2):
        ba1 = (bx2 - bx1) * (by2 - by1)
        def blk(k, carry):
            rv, ri = carry
            sl = pl.ds(k * _L, _L)
            s = cs_v[sl]
            x1 = cx1_v[sl]; y1 = cy1_v[sl]; x2 = cx2_v[sl]; y2 = cy2_v[sl]
            ltx = jnp.maximum(bx1, x1); lty = jnp.maximum(by1, y1)
            rbx = jnp.minimum(bx2, x2); rby = jnp.minimum(by2, y2)
            w = jnp.maximum(rbx - ltx, 0.0); h = jnp.maximum(rby - lty, 0.0)
            inter = w * h
            a2 = ca_v[sl]
            iou = inter / (ba1 + a2 - inter + jnp.float32(1e-8))
            s2 = jnp.where(iou > _IOU_THR, -1.0, s)
            cs_v[sl] = s2
            idxv = k * _L + lanes
            upd = s2 > rv
            ri = jnp.where(upd, idxv, ri)
            rv = jnp.maximum(rv, s2)
            return (rv, ri)
        rv0 = jnp.full((_L,), -2.0, jnp.float32)
        rv, ri = lax.fori_loop(0, _CCH, blk, (rv0, zero), unroll=4)
        mval = jnp.max(rv)
        midx = jnp.min(jnp.where(rv == mval, ri, jnp.int32(_CAND)))
        return mval, midx

    far = jnp.full((_L,), -1e6, jnp.float32)
    mval, midx = nms_pass(far, far, far, far)

    def init_o(k, _):
        sl = pl.ds(k * _L, _L)
        os_v[sl] = jnp.zeros((_L,), jnp.float32)
        for comp in range(4):
            ob_v[comp, sl] = jnp.zeros((_L,), jnp.float32)
        return 0
    lax.fori_loop(0, _SEL // _L, init_o, 0)

    def sel_body(i, carry):
        mval, midx = carry
        iv = jnp.full((_L,), midx, jnp.int32)
        valid = mval > 0.0
        bx1 = plsc.load_gather(cx1_v, [iv])
        by1 = plsc.load_gather(cy1_v, [iv])
        bx2 = plsc.load_gather(cx2_v, [iv])
        by2 = plsc.load_gather(cy2_v, [iv])
        tv = jnp.full((_L,), i, jnp.int32)
        sval = jnp.where(valid, mval, 0.0)
        plsc.store_scatter(os_v, [tv], jnp.full((_L,), sval, jnp.float32),
                           mask=lane0)
        vf = jnp.where(valid, 1.0, 0.0)
        plsc.store_scatter(ob_v, [jnp.zeros((_L,), jnp.int32), tv],
                           bx1 * vf, mask=lane0)
        plsc.store_scatter(ob_v, [jnp.full((_L,), 1, jnp.int32), tv],
                           by1 * vf, mask=lane0)
        plsc.store_scatter(ob_v, [jnp.full((_L,), 2, jnp.int32), tv],
                           bx2 * vf, mask=lane0)
        plsc.store_scatter(ob_v, [jnp.full((_L,), 3, jnp.int32), tv],
                           by2 * vf, mask=lane0)
        plsc.store_scatter(cs_v, [iv], jnp.full((_L,), -1.0, jnp.float32),
                           mask=lane0)
        return nms_pass(bx1, by1, bx2, by2)

    lax.fori_loop(0, _MAX_TOTAL, sel_body, (mval, midx))

    pltpu.sync_copy(os_v, sel_s_hbm.at[pl.ds(p * _SEL, _SEL)])
    for comp in range(4):
        pltpu.sync_copy(ob_v.at[comp],
                        sel_b_hbm.at[pl.ds((p * 4 + comp) * _SEL, _SEL)])


@functools.partial(
    pl.kernel,
    mesh=_mesh,
    out_type=(
        jax.ShapeDtypeStruct((_NPROB * _SEL,), jnp.float32),
        jax.ShapeDtypeStruct((_NPROB * 4 * _SEL,), jnp.float32),
    ),
    compiler_params=pltpu.CompilerParams(needs_layout_passes=False),
    scratch_types=[
        pltpu.VMEM((128,), jnp.float32),
        pltpu.VMEM((128,), jnp.int32),
        pltpu.VMEM((_NPAD,), jnp.float32),
        pltpu.VMEM((_CAND,), jnp.int32),
        pltpu.VMEM((_CAND,), jnp.float32),
        pltpu.VMEM((_CAND,), jnp.float32),
        pltpu.VMEM((_CAND,), jnp.float32),
        pltpu.VMEM((_CAND,), jnp.float32),
        pltpu.VMEM((_CAND,), jnp.float32),
        pltpu.VMEM((_CAND,), jnp.float32),
        pltpu.VMEM((_SEL,), jnp.float32),
        pltpu.VMEM((4, _SEL), jnp.float32),
        pltpu.SemaphoreType.DMA,
    ],
)
def _sc_nms(scores_hbm, boxes_hbm, thr_hbm, quota_hbm,
            sel_s_hbm, sel_b_hbm,
            thr_v, quota_v, s_v, cidx_v, cs_v, cx1_v, cy1_v, cx2_v, cy2_v,
            ca_v, os_v, ob_v, sem):
    wid = lax.axis_index("s") * 2 + lax.axis_index("c")
    pltpu.sync_copy(thr_hbm, thr_v)
    pltpu.sync_copy(quota_hbm, quota_v)
    args = (scores_hbm, boxes_hbm, thr_v, quota_v, s_v, cidx_v, cs_v,
            cx1_v, cy1_v, cx2_v, cy2_v, ca_v, os_v, ob_v, sel_s_hbm,
            sel_b_hbm, sem)
    _sc_nms_problem(wid, *args)

    @pl.when(wid < _NPROB - 32)
    def _():
        _sc_nms_problem(wid + 32, *args)


@functools.partial(
    pl.kernel,
    mesh=_mesh,
    out_type=(
        jax.ShapeDtypeStruct((_BATCH * _SEL,), jnp.float32),
        jax.ShapeDtypeStruct((_BATCH * 4 * _SEL,), jnp.float32),
        jax.ShapeDtypeStruct((_BATCH * _SEL,), jnp.float32),
    ),
    compiler_params=pltpu.CompilerParams(needs_layout_passes=False),
    scratch_types=[
        pltpu.VMEM((_NUM_CLASSES * _SEL,), jnp.float32),
        pltpu.VMEM((_NUM_CLASSES * 4 * _SEL,), jnp.float32),
        pltpu.VMEM((_SEL,), jnp.float32),
        pltpu.VMEM((4, _SEL), jnp.float32),
        pltpu.VMEM((_SEL,), jnp.float32),
    ],
)
def _sc_merge(sel_s_hbm, sel_b_hbm, out_s_hbm, out_b_hbm, out_l_hbm,
              ss_v, sb_v, os_v, ob_v, ol_v):
    wid = lax.axis_index("s") * 2 + lax.axis_index("c")
    lanes = lax.iota(jnp.int32, _L)
    lane0 = lanes == 0
    big = jnp.int32(9999)

    @pl.when(wid < _BATCH)
    def _():
        b = wid
        pltpu.sync_copy(
            sel_s_hbm.at[pl.ds(b * _NUM_CLASSES * _SEL, _NUM_CLASSES * _SEL)],
            ss_v)
        pltpu.sync_copy(
            sel_b_hbm.at[pl.ds(b * _NUM_CLASSES * 4 * _SEL,
                               _NUM_CLASSES * 4 * _SEL)],
            sb_v)

        def init_o(k, _):
            sl = pl.ds(k * _L, _L)
            os_v[sl] = jnp.zeros((_L,), jnp.float32)
            ol_v[sl] = jnp.zeros((_L,), jnp.float32)
            for comp in range(4):
                ob_v[comp, sl] = jnp.zeros((_L,), jnp.float32)
            return 0
        lax.fori_loop(0, _SEL // _L, init_o, 0)

        cls0 = lanes
        cls1 = _L + lanes
        live1 = cls1 < _NUM_CLASSES

        def step(t, carry):
            h0, h1 = carry
            v0 = plsc.load_gather(
                ss_v, [cls0 * _SEL + jnp.minimum(h0, _SEL - 1)])
            v1 = plsc.load_gather(
                ss_v, [jnp.minimum(cls1, _NUM_CLASSES - 1) * _SEL
                       + jnp.minimum(h1, _SEL - 1)])
            v0 = jnp.where(h0 >= _MAX_TOTAL, -1.0, v0)
            v1 = jnp.where(live1 & (h1 < _MAX_TOTAL), v1, -1.0)
            m0 = jnp.max(v0)
            m1 = jnp.max(v1)
            mval = jnp.maximum(m0, m1)
            c0 = jnp.min(jnp.where(v0 == mval, cls0, big))
            c1 = jnp.min(jnp.where(v1 == mval, cls1, big))
            cls = jnp.minimum(c0, c1)
            p0 = jnp.min(jnp.where((v0 == mval) & (cls0 == cls), h0, big))
            p1 = jnp.min(jnp.where((v1 == mval) & (cls1 == cls), h1, big))
            pos = jnp.minimum(p0, p1)

            cv = jnp.full((_L,), cls, jnp.int32)
            pv = jnp.full((_L,), pos, jnp.int32)
            tv = jnp.full((_L,), t, jnp.int32)
            plsc.store_scatter(os_v, [tv], jnp.full((_L,), mval, jnp.float32),
                               mask=lane0)
            lab = jnp.where(mval > 0.0, cls.astype(jnp.float32), 0.0)
            plsc.store_scatter(ol_v, [tv], jnp.full((_L,), lab, jnp.float32),
                               mask=lane0)
            for comp in range(4):
                bv = plsc.load_gather(
                    sb_v, [cv * (4 * _SEL) + comp * _SEL + pv])
                plsc.store_scatter(ob_v,
                                   [jnp.full((_L,), comp, jnp.int32), tv],
                                   bv, mask=lane0)
            adv0 = (cls0 == cls)
            adv1 = (cls1 == cls)
            h0 = h0 + jnp.where(adv0, 1, 0)
            h1 = h1 + jnp.where(adv1, 1, 0)
            return (h0, h1)

        zero = jnp.zeros((_L,), jnp.int32)
        lax.fori_loop(0, _MAX_TOTAL, step, (zero, zero))

        pltpu.sync_copy(os_v, out_s_hbm.at[pl.ds(b * _SEL, _SEL)])
        pltpu.sync_copy(ol_v, out_l_hbm.at[pl.ds(b * _SEL, _SEL)])
        for comp in range(4):
            pltpu.sync_copy(
                ob_v.at[comp],
                out_b_hbm.at[pl.ds((b * 4 + comp) * _SEL, _SEL)])


@jax.jit
def _run(box_pred, cls_pred):
    anch = jnp.asarray(_ANCH)
    logits_t = jnp.transpose(cls_pred, (0, 2, 1)).reshape(_NPROB, _N)
    logits_t = jnp.pad(logits_t, ((0, 0), (0, _NPAD - _N)),
                       constant_values=-1e9)
    bp_t = jnp.transpose(box_pred, (0, 2, 1))
    bp_t = jnp.pad(bp_t, ((0, 0), (0, 0), (0, _NPAD - _N)))

    scores, boxes, thr, quota = _run_tc(logits_t, bp_t, anch)
    thr_pad = jnp.pad(thr[:, 0], (0, 128 - _NPROB))
    quota_pad = jnp.pad(quota[:, 0], (0, 128 - _NPROB))
    sel_s, sel_b = _sc_nms(scores, boxes, thr_pad, quota_pad)
    out_s, out_b, out_l = _sc_merge(sel_s, sel_b)

    ts = out_s.reshape(_BATCH, _SEL)[:, :_MAX_TOTAL]
    tl = out_l.reshape(_BATCH, _SEL)[:, :_MAX_TOTAL]
    tb = jnp.transpose(out_b.reshape(_BATCH, 4, _SEL),
                       (0, 2, 1))[:, :_MAX_TOTAL, :]
    return (tb, ts, tl)


def kernel(box_pred, cls_pred):
    return _run(box_pred, cls_pred)


# speedup vs baseline: 21.5015x; 1.0072x over previous
import functools

import jax
import jax.numpy as jnp
import numpy as np
from jax import lax
from jax.experimental import pallas as pl
from jax.experimental.pallas import tpu as pltpu
from jax.experimental.pallas import tpu_sc as plsc

_NUM_CLASSES = 20
_BATCH = 2
_MAX_TOTAL = 200
_SCORE_THR = 0.5
_IOU_THR = 0.5
_PRE_NMS = 2000
_N = 49104
_NPAD = 49152
_NPROB = _BATCH * _NUM_CLASSES
_CAND = 2048
_SEL = 256


def _make_anchors(image_h, image_w):
    aspect_ratios = [0.5, 1.0, 2.0]
    scales = [2.0 ** x for x in [0.0, 1.0 / 3.0, 2.0 / 3.0]]
    areas = [32.0 ** 2, 64.0 ** 2, 128.0 ** 2, 256.0 ** 2, 512.0 ** 2]
    strides = [2 ** i for i in range(3, 8)]
    all_anchors = []
    for area, stride in zip(areas, strides):
        dims = []
        for ratio in aspect_ratios:
            ah = np.sqrt(area / ratio)
            aw = area / ah
            for scale in scales:
                dims.append([aw * scale, ah * scale])
        dims = np.array(dims, dtype=np.float32)
        fh = int(np.ceil(image_h / stride))
        fw = int(np.ceil(image_w / stride))
        rx = (np.arange(fw, dtype=np.float32) + 0.5) * stride
        ry = (np.arange(fh, dtype=np.float32) + 0.5) * stride
        cx, cy = np.meshgrid(rx, ry)
        centers = np.stack([cx, cy], axis=-1).astype(np.float32)[:, :, None, :]
        centers = np.tile(centers, (1, 1, 9, 1))
        d = np.broadcast_to(dims[None, None, :, :], (fh, fw, 9, 2))
        all_anchors.append(np.concatenate([centers, d], axis=-1).reshape(-1, 4))
    return np.concatenate(all_anchors, axis=0)


_ANCH = np.zeros((4, _NPAD), dtype=np.float32)
_ANCH[:, :_N] = _make_anchors(512, 512).T
_ANCH[2:, _N:] = 1.0


_BLK = 512


def _tc_body(logits_ref, bp_ref, anch_ref, scores_ref, boxes_ref,
             thr_ref, quota_ref):
    nblk = _NPAD // _BLK

    def sig_blk(j, _):
        s = pl.ds(j * _BLK, _BLK)
        x = logits_ref[:, s]
        scores_ref[:, s] = 1.0 / (1.0 + jnp.exp(-x))
        return 0
    lax.fori_loop(0, nblk, sig_blk, 0, unroll=2)

    def dec_blk(j, _):
        s = pl.ds(j * _BLK, _BLK)
        bp = bp_ref[:, :, s]
        an = anch_ref[:, s]
        acx, acy, aw, ah = an[0], an[1], an[2], an[3]
        tx = bp[:, 0] * jnp.float32(0.1)
        ty = bp[:, 1] * jnp.float32(0.1)
        tw = bp[:, 2] * jnp.float32(0.2)
        th = bp[:, 3] * jnp.float32(0.2)
        cx = tx * aw + acx
        cy = ty * ah + acy
        w = jnp.exp(tw) * aw
        h = jnp.exp(th) * ah
        wh_x = w / 2.0
        wh_y = h / 2.0
        boxes_ref[:, 0, s] = cx - wh_x
        boxes_ref[:, 1, s] = cy - wh_y
        boxes_ref[:, 2, s] = cx + wh_x
        boxes_ref[:, 3, s] = cy + wh_y
        return 0
    lax.fori_loop(0, nblk, dec_blk, 0, unroll=2)

    def count_ge(test):
        def blk(j, acc):
            s = pl.ds(j * _BLK, _BLK)
            b = lax.bitcast_convert_type(scores_ref[:, s], jnp.int32)
            return acc + jnp.sum(
                jnp.where(b >= test, jnp.int32(1), jnp.int32(0)),
                axis=1, keepdims=True)
        return lax.fori_loop(0, nblk, blk, jnp.zeros((_NPROB, 1), jnp.int32),
                             unroll=2)

    def round_body(r, p):
        test = p + lax.shift_left(jnp.int32(1), jnp.int32(29) - r)
        cnt = count_ge(test)
        return jnp.where(cnt >= _PRE_NMS, test, p)

    pbits = lax.fori_loop(0, 30, round_body, jnp.zeros((_NPROB, 1), jnp.int32))
    cnt_gt = count_ge(pbits + 1)
    quota = _PRE_NMS - cnt_gt

    thr_ref[:, :] = lax.bitcast_convert_type(pbits, jnp.float32)
    quota_ref[:, :] = quota


def _run_tc(logits_t, bp_t, anch):
    return pl.pallas_call(
        _tc_body,
        out_shape=(
            jax.ShapeDtypeStruct((_NPROB, _NPAD), jnp.float32),
            jax.ShapeDtypeStruct((_BATCH, 4, _NPAD), jnp.float32),
            jax.ShapeDtypeStruct((_NPROB, 1), jnp.float32),
            jax.ShapeDtypeStruct((_NPROB, 1), jnp.int32),
        ),
    )(logits_t, bp_t, anch)


_mesh = plsc.VectorSubcoreMesh(core_axis_name="c", subcore_axis_name="s")
_L = 16
_NCH = _NPAD // _L
_CCH = _PRE_NMS // _L
_CCHF = _CAND // _L


def _sc_nms_problem(p, scores_hbm, boxes_hbm, thr_v, quota_v,
                    s_v, cidx_v, cs_v, cx1_v, cy1_v, cx2_v, cy2_v,
                    ca_v, os_v, ob_v, sel_s_hbm, sel_b_hbm, sem):
    lanes = lax.iota(jnp.int32, _L)
    lane0 = lanes == 0
    p_vec = jnp.full((_L,), p, jnp.int32)
    b = p // _NUM_CLASSES

    pltpu.sync_copy(scores_hbm.at[p], s_v)
    thr = plsc.load_gather(thr_v, [p_vec])
    quota = plsc.load_gather(quota_v, [p_vec])

    def init_s(k, _):
        cs_v[pl.ds(k * _L, _L)] = jnp.full((_L,), -1.0, jnp.float32)
        return 0
    lax.fori_loop(0, _CCHF, init_s, 0, unroll=4)
    def init_i(k, _):
        cidx_v[pl.ds(k * _L, _L)] = jnp.zeros((_L,), jnp.int32)
        return 0
    lax.fori_loop(0, _CCHF, init_i, 0, unroll=4)

    def comp_blk(k, carry):
        ptr, ties = carry
        sl = pl.ds(k * _L, _L)
        s = s_v[sl]
        idxv = k * _L + lanes
        gt = s > thr
        eq = s == thr
        pref = jnp.cumsum(eq.astype(jnp.int32))
        tie_ok = eq & ((ties + pref - 1) < quota)
        m = (s >= _SCORE_THR) & (gt | tie_ok)
        pos = jnp.cumsum(m.astype(jnp.int32))
        dst = ptr + pos - 1
        plsc.store_scatter(cs_v, [dst], s, mask=m)
        plsc.store_scatter(cidx_v, [dst], idxv, mask=m)
        ptr = ptr + plsc.all_reduce_population_count(m)
        ties = ties + plsc.all_reduce_population_count(eq)
        return (ptr, ties)

    zero = jnp.zeros((_L,), jnp.int32)
    lax.fori_loop(0, _NCH, comp_blk, (zero, zero), unroll=4)

    for comp, dstv in enumerate((cx1_v, cy1_v, cx2_v, cy2_v)):
        pltpu.sync_copy(boxes_hbm.at[b, comp], s_v)
        def gat(k, _):
            sl = pl.ds(k * _L, _L)
            dstv[sl] = plsc.load_gather(s_v, [cidx_v[sl]])
            return 0
        lax.fori_loop(0, _CCH, gat, 0, unroll=5)

    def area_blk(k, _):
        sl = pl.ds(k * _L, _L)
        ca_v[sl] = (cx2_v[sl] - cx1_v[sl]) * (cy2_v[sl] - cy1_v[sl])
        return 0
    lax.fori_loop(0, _CCH, area_blk, 0, unroll=5)

    def nms_pass(bx1, by1, bx2, by2):
        ba1 = (bx2 - bx1) * (by2 - by1)
        def blk(k, carry):
            rv, ri = carry
            sl = pl.ds(k * _L, _L)
            s = cs_v[sl]
            x1 = cx1_v[sl]; y1 = cy1_v[sl]; x2 = cx2_v[sl]; y2 = cy2_v[sl]
            ltx = jnp.maximum(bx1, x1); lty = jnp.maximum(by1, y1)
            rbx = jnp.minimum(bx2, x2); rby = jnp.minimum(by2, y2)
            w = jnp.maximum(rbx - ltx, 0.0); h = jnp.maximum(rby - lty, 0.0)
            inter = w * h
            a2 = ca_v[sl]
            iou = inter / (ba1 + a2 - inter + jnp.float32(1e-8))
            s2 = jnp.where(iou > _IOU_THR, -1.0, s)
            cs_v[sl] = s2
            idxv = k * _L + lanes
            upd = s2 > rv
            ri = jnp.where(upd, idxv, ri)
            rv = jnp.maximum(rv, s2)
            return (rv, ri)
        rv0 = jnp.full((_L,), -2.0, jnp.float32)
        rv, ri = lax.fori_loop(0, _CCH, blk, (rv0, zero), unroll=5)
        mval = jnp.max(rv)
        midx = jnp.min(jnp.where(rv == mval, ri, jnp.int32(_CAND)))
        return mval, midx

    far = jnp.full((_L,), -1e6, jnp.float32)
    mval, midx = nms_pass(far, far, far, far)

    def init_o(k, _):
        sl = pl.ds(k * _L, _L)
        os_v[sl] = jnp.zeros((_L,), jnp.float32)
        for comp in range(4):
            ob_v[comp, sl] = jnp.zeros((_L,), jnp.float32)
        return 0
    lax.fori_loop(0, _SEL // _L, init_o, 0)

    def sel_body(i, carry):
        mval, midx = carry
        iv = jnp.full((_L,), midx, jnp.int32)
        valid = mval > 0.0
        bx1 = plsc.load_gather(cx1_v, [iv])
        by1 = plsc.load_gather(cy1_v, [iv])
        bx2 = plsc.load_gather(cx2_v, [iv])
        by2 = plsc.load_gather(cy2_v, [iv])
        tv = jnp.full((_L,), i, jnp.int32)
        sval = jnp.where(valid, mval, 0.0)
        plsc.store_scatter(os_v, [tv], jnp.full((_L,), sval, jnp.float32),
                           mask=lane0)
        vf = jnp.where(valid, 1.0, 0.0)
        plsc.store_scatter(ob_v, [jnp.zeros((_L,), jnp.int32), tv],
                           bx1 * vf, mask=lane0)
        plsc.store_scatter(ob_v, [jnp.full((_L,), 1, jnp.int32), tv],
                           by1 * vf, mask=lane0)
        plsc.store_scatter(ob_v, [jnp.full((_L,), 2, jnp.int32), tv],
                           bx2 * vf, mask=lane0)
        plsc.store_scatter(ob_v, [jnp.full((_L,), 3, jnp.int32), tv],
                           by2 * vf, mask=lane0)
        plsc.store_scatter(cs_v, [iv], jnp.full((_L,), -1.0, jnp.float32),
                           mask=lane0)
        return nms_pass(bx1, by1, bx2, by2)

    lax.fori_loop(0, _MAX_TOTAL, sel_body, (mval, midx))

    pltpu.sync_copy(os_v, sel_s_hbm.at[pl.ds(p * _SEL, _SEL)])
    for comp in range(4):
        pltpu.sync_copy(ob_v.at[comp],
                        sel_b_hbm.at[pl.ds((p * 4 + comp) * _SEL, _SEL)])


@functools.partial(
    pl.kernel,
    mesh=_mesh,
    out_type=(
        jax.ShapeDtypeStruct((_NPROB * _SEL,), jnp.float32),
        jax.ShapeDtypeStruct((_NPROB * 4 * _SEL,), jnp.float32),
    ),
    compiler_params=pltpu.CompilerParams(needs_layout_passes=False),
    scratch_types=[
        pltpu.VMEM((128,), jnp.float32),
        pltpu.VMEM((128,), jnp.int32),
        pltpu.VMEM((_NPAD,), jnp.float32),
        pltpu.VMEM((_CAND,), jnp.int32),
        pltpu.VMEM((_CAND,), jnp.float32),
        pltpu.VMEM((_CAND,), jnp.float32),
        pltpu.VMEM((_CAND,), jnp.float32),
        pltpu.VMEM((_CAND,), jnp.float32),
        pltpu.VMEM((_CAND,), jnp.float32),
        pltpu.VMEM((_CAND,), jnp.float32),
        pltpu.VMEM((_SEL,), jnp.float32),
        pltpu.VMEM((4, _SEL), jnp.float32),
        pltpu.SemaphoreType.DMA,
    ],
)
def _sc_nms(scores_hbm, boxes_hbm, thr_hbm, quota_hbm,
            sel_s_hbm, sel_b_hbm,
            thr_v, quota_v, s_v, cidx_v, cs_v, cx1_v, cy1_v, cx2_v, cy2_v,
            ca_v, os_v, ob_v, sem):
    wid = lax.axis_index("s") * 2 + lax.axis_index("c")
    pltpu.sync_copy(thr_hbm, thr_v)
    pltpu.sync_copy(quota_hbm, quota_v)
    args = (scores_hbm, boxes_hbm, thr_v, quota_v, s_v, cidx_v, cs_v,
            cx1_v, cy1_v, cx2_v, cy2_v, ca_v, os_v, ob_v, sel_s_hbm,
            sel_b_hbm, sem)
    _sc_nms_problem(wid, *args)

    @pl.when(wid < _NPROB - 32)
    def _():
        _sc_nms_problem(wid + 32, *args)


@functools.partial(
    pl.kernel,
    mesh=_mesh,
    out_type=(
        jax.ShapeDtypeStruct((_BATCH * _SEL,), jnp.float32),
        jax.ShapeDtypeStruct((_BATCH * 4 * _SEL,), jnp.float32),
        jax.ShapeDtypeStruct((_BATCH * _SEL,), jnp.float32),
    ),
    compiler_params=pltpu.CompilerParams(needs_layout_passes=False),
    scratch_types=[
        pltpu.VMEM((_NUM_CLASSES * _SEL,), jnp.float32),
        pltpu.VMEM((_NUM_CLASSES * 4 * _SEL,), jnp.float32),
        pltpu.VMEM((_SEL,), jnp.float32),
        pltpu.VMEM((4, _SEL), jnp.float32),
        pltpu.VMEM((_SEL,), jnp.float32),
    ],
)
def _sc_merge(sel_s_hbm, sel_b_hbm, out_s_hbm, out_b_hbm, out_l_hbm,
              ss_v, sb_v, os_v, ob_v, ol_v):
    wid = lax.axis_index("s") * 2 + lax.axis_index("c")
    lanes = lax.iota(jnp.int32, _L)
    lane0 = lanes == 0
    big = jnp.int32(9999)

    @pl.when(wid < _BATCH)
    def _():
        b = wid
        pltpu.sync_copy(
            sel_s_hbm.at[pl.ds(b * _NUM_CLASSES * _SEL, _NUM_CLASSES * _SEL)],
            ss_v)
        pltpu.sync_copy(
            sel_b_hbm.at[pl.ds(b * _NUM_CLASSES * 4 * _SEL,
                               _NUM_CLASSES * 4 * _SEL)],
            sb_v)

        def init_o(k, _):
            sl = pl.ds(k * _L, _L)
            os_v[sl] = jnp.zeros((_L,), jnp.float32)
            ol_v[sl] = jnp.zeros((_L,), jnp.float32)
            for comp in range(4):
                ob_v[comp, sl] = jnp.zeros((_L,), jnp.float32)
            return 0
        lax.fori_loop(0, _SEL // _L, init_o, 0)

        cls0 = lanes
        cls1 = _L + lanes
        live1 = cls1 < _NUM_CLASSES

        def step(t, carry):
            h0, h1 = carry
            v0 = plsc.load_gather(
                ss_v, [cls0 * _SEL + jnp.minimum(h0, _SEL - 1)])
            v1 = plsc.load_gather(
                ss_v, [jnp.minimum(cls1, _NUM_CLASSES - 1) * _SEL
                       + jnp.minimum(h1, _SEL - 1)])
            v0 = jnp.where(h0 >= _MAX_TOTAL, -1.0, v0)
            v1 = jnp.where(live1 & (h1 < _MAX_TOTAL), v1, -1.0)
            m0 = jnp.max(v0)
            m1 = jnp.max(v1)
            mval = jnp.maximum(m0, m1)
            c0 = jnp.min(jnp.where(v0 == mval, cls0, big))
            c1 = jnp.min(jnp.where(v1 == mval, cls1, big))
            cls = jnp.minimum(c0, c1)
            p0 = jnp.min(jnp.where((v0 == mval) & (cls0 == cls), h0, big))
            p1 = jnp.min(jnp.where((v1 == mval) & (cls1 == cls), h1, big))
            pos = jnp.minimum(p0, p1)

            cv = jnp.full((_L,), cls, jnp.int32)
            pv = jnp.full((_L,), pos, jnp.int32)
            tv = jnp.full((_L,), t, jnp.int32)
            plsc.store_scatter(os_v, [tv], jnp.full((_L,), mval, jnp.float32),
                               mask=lane0)
            lab = jnp.where(mval > 0.0, cls.astype(jnp.float32), 0.0)
            plsc.store_scatter(ol_v, [tv], jnp.full((_L,), lab, jnp.float32),
                               mask=lane0)
            for comp in range(4):
                bv = plsc.load_gather(
                    sb_v, [cv * (4 * _SEL) + comp * _SEL + pv])
                plsc.store_scatter(ob_v,
                                   [jnp.full((_L,), comp, jnp.int32), tv],
                                   bv, mask=lane0)
            adv0 = (cls0 == cls)
            adv1 = (cls1 == cls)
            h0 = h0 + jnp.where(adv0, 1, 0)
            h1 = h1 + jnp.where(adv1, 1, 0)
            return (h0, h1)

        zero = jnp.zeros((_L,), jnp.int32)
        lax.fori_loop(0, _MAX_TOTAL, step, (zero, zero))

        pltpu.sync_copy(os_v, out_s_hbm.at[pl.ds(b * _SEL, _SEL)])
        pltpu.sync_copy(ol_v, out_l_hbm.at[pl.ds(b * _SEL, _SEL)])
        for comp in range(4):
            pltpu.sync_copy(
                ob_v.at[comp],
                out_b_hbm.at[pl.ds((b * 4 + comp) * _SEL, _SEL)])


@jax.jit
def _run(box_pred, cls_pred):
    anch = jnp.asarray(_ANCH)
    logits_t = jnp.transpose(cls_pred, (0, 2, 1)).reshape(_NPROB, _N)
    logits_t = jnp.pad(logits_t, ((0, 0), (0, _NPAD - _N)),
                       constant_values=-1e9)
    bp_t = jnp.transpose(box_pred, (0, 2, 1))
    bp_t = jnp.pad(bp_t, ((0, 0), (0, 0), (0, _NPAD - _N)))

    scores, boxes, thr, quota = _run_tc(logits_t, bp_t, anch)
    thr_pad = jnp.pad(thr[:, 0], (0, 128 - _NPROB))
    quota_pad = jnp.pad(quota[:, 0], (0, 128 - _NPROB))
    sel_s, sel_b = _sc_nms(scores, boxes, thr_pad, quota_pad)
    out_s, out_b, out_l = _sc_merge(sel_s, sel_b)

    ts = out_s.reshape(_BATCH, _SEL)[:, :_MAX_TOTAL]
    tl = out_l.reshape(_BATCH, _SEL)[:, :_MAX_TOTAL]
    tb = jnp.transpose(out_b.reshape(_BATCH, 4, _SEL),
                       (0, 2, 1))[:, :_MAX_TOTAL, :]
    return (tb, ts, tl)


def kernel(box_pred, cls_pred):
    return _run(box_pred, cls_pred)
